# Initial kernel scaffold; baseline (speedup 1.0000x reference)
import jax
import jax.numpy as jnp
from jax.experimental import pallas as pl


def kernel(x, edge_index, edge_attr, W_e, b_e, W_s, b_s, W_t, b_t, W_1, b_1):
    raise NotImplementedError("write your pallas kernel here")


# speedup vs baseline: 2.4342x; 2.4342x over previous
import functools

import jax
import jax.numpy as jnp
from jax import lax
from jax.experimental import pallas as pl
from jax.experimental.pallas import tpu as pltpu
from jax.experimental.pallas import tpu_sc as plsc


def _node_body(x_ref, ws_ref, bs_ref, wt_ref, bt_ref, xs_ref, xt_ref):
    xv = x_ref[...]
    xs_ref[...] = jnp.dot(xv, ws_ref[...], preferred_element_type=jnp.float32) + bs_ref[...]
    xt_ref[...] = jnp.dot(xv, wt_ref[...], preferred_element_type=jnp.float32) + bt_ref[...]


@functools.partial(jax.jit, static_argnames=())
def _node_transform(x, ws_t, bs, wt_t, bt):
    n, d = x.shape
    de = ws_t.shape[1]
    blk = 1000
    grid = n // blk
    return pl.pallas_call(
        _node_body,
        grid=(grid,),
        in_specs=[
            pl.BlockSpec((blk, d), lambda i: (i, 0)),
            pl.BlockSpec((d, de), lambda i: (0, 0)),
            pl.BlockSpec((1, de), lambda i: (0, 0)),
            pl.BlockSpec((d, de), lambda i: (0, 0)),
            pl.BlockSpec((1, de), lambda i: (0, 0)),
        ],
        out_specs=[
            pl.BlockSpec((blk, de), lambda i: (i, 0)),
            pl.BlockSpec((blk, de), lambda i: (i, 0)),
        ],
        out_shape=[
            jax.ShapeDtypeStruct((n, de), jnp.float32),
            jax.ShapeDtypeStruct((n, de), jnp.float32),
        ],
    )(x, ws_t, bs.reshape(1, de), wt_t, bt.reshape(1, de))


def _make_gather(n_edges, d):
    info = plsc.get_sparse_core_info()
    nc, ns = info.num_cores, info.num_subcores
    nw = nc * ns
    b_per_w = n_edges // nw
    chunk = 80
    n_chunks = b_per_w // chunk
    mesh = plsc.VectorSubcoreMesh(core_axis_name="c", subcore_axis_name="s")

    @functools.partial(
        pl.kernel,
        mesh=mesh,
        out_type=[
            jax.ShapeDtypeStruct((n_edges, d), jnp.float32),
            jax.ShapeDtypeStruct((n_edges, d), jnp.float32),
        ],
        scratch_types=[
            pltpu.VMEM((chunk,), jnp.int32),
            pltpu.VMEM((chunk,), jnp.int32),
            pltpu.VMEM((chunk, d), jnp.float32),
            pltpu.VMEM((chunk, d), jnp.float32),
            pltpu.SemaphoreType.DMA,
            pltpu.SemaphoreType.DMA,
        ],
    )
    def gather_k(xs_hbm, xt_hbm, src_hbm, dst_hbm, gs_hbm, gt_hbm,
                 idx_s, idx_t, rows_s, rows_t, sem_s, sem_t):
        wid = lax.axis_index("s") * nc + lax.axis_index("c")
        base = wid * b_per_w

        def body(ci, carry):
            off = base + ci * chunk
            pltpu.sync_copy(src_hbm.at[pl.ds(off, chunk)], idx_s)
            pltpu.sync_copy(dst_hbm.at[pl.ds(off, chunk)], idx_t)
            cp1 = pltpu.async_copy(xs_hbm.at[idx_s], rows_s, sem_s)
            cp2 = pltpu.async_copy(xt_hbm.at[idx_t], rows_t, sem_t)
            cp1.wait()
            cp2.wait()
            pltpu.sync_copy(rows_s, gs_hbm.at[pl.ds(off, chunk)])
            pltpu.sync_copy(rows_t, gt_hbm.at[pl.ds(off, chunk)])
            return carry

        lax.fori_loop(0, n_chunks, body, 0)

    return gather_k


def _edge_body(ea_ref, gs_ref, gt_ref, we_ref, be_ref, w1_ref, b1_ref, out_ref):
    h = jnp.dot(ea_ref[...], we_ref[...], preferred_element_type=jnp.float32)
    h = h + be_ref[...] + gs_ref[...] + gt_ref[...]
    h = jnp.maximum(h, 0.0)
    out_ref[...] = jnp.dot(h, w1_ref[...], preferred_element_type=jnp.float32) + b1_ref[...]


def _edge_mlp(ea, gs, gt, we_t, be, w1_t, b1):
    e, d = ea.shape
    blk = 2560
    grid = e // blk
    return pl.pallas_call(
        _edge_body,
        grid=(grid,),
        in_specs=[
            pl.BlockSpec((blk, d), lambda i: (i, 0)),
            pl.BlockSpec((blk, d), lambda i: (i, 0)),
            pl.BlockSpec((blk, d), lambda i: (i, 0)),
            pl.BlockSpec((d, d), lambda i: (0, 0)),
            pl.BlockSpec((1, d), lambda i: (0, 0)),
            pl.BlockSpec((d, d), lambda i: (0, 0)),
            pl.BlockSpec((1, d), lambda i: (0, 0)),
        ],
        out_specs=pl.BlockSpec((blk, d), lambda i: (i, 0)),
        out_shape=jax.ShapeDtypeStruct((e, d), jnp.float32),
    )(ea, gs, gt, we_t, be.reshape(1, d), w1_t, b1.reshape(1, d))


def kernel(x, edge_index, edge_attr, W_e, b_e, W_s, b_s, W_t, b_t, W_1, b_1):
    src = edge_index[0].astype(jnp.int32)
    dst = edge_index[1].astype(jnp.int32)
    xs, xt = _node_transform(x, W_s.T, b_s, W_t.T, b_t)
    gather = _make_gather(edge_attr.shape[0], edge_attr.shape[1])
    gs, gt = gather(xs, xt, src, dst)
    return _edge_mlp(edge_attr, gs, gt, W_e.T, b_e, W_1.T, b_1)


# speedup vs baseline: 3.2256x; 1.3251x over previous
import functools

import jax
import jax.numpy as jnp
from jax import lax
from jax.experimental import pallas as pl
from jax.experimental.pallas import tpu as pltpu
from jax.experimental.pallas import tpu_sc as plsc


def _node_body(x_ref, ws_ref, bs_ref, wt_ref, bt_ref, xs_ref, xt_ref):
    xv = x_ref[...]
    xs_ref[...] = jnp.dot(xv, ws_ref[...], preferred_element_type=jnp.float32) + bs_ref[...]
    xt_ref[...] = jnp.dot(xv, wt_ref[...], preferred_element_type=jnp.float32) + bt_ref[...]


def _node_transform(x, ws_t, bs, wt_t, bt):
    n, d = x.shape
    de = ws_t.shape[1]
    blk = 1000
    grid = n // blk
    return pl.pallas_call(
        _node_body,
        grid=(grid,),
        in_specs=[
            pl.BlockSpec((blk, d), lambda i: (i, 0)),
            pl.BlockSpec((d, de), lambda i: (0, 0)),
            pl.BlockSpec((1, de), lambda i: (0, 0)),
            pl.BlockSpec((d, de), lambda i: (0, 0)),
            pl.BlockSpec((1, de), lambda i: (0, 0)),
        ],
        out_specs=[
            pl.BlockSpec((blk, de), lambda i: (i, 0)),
            pl.BlockSpec((blk, de), lambda i: (i, 0)),
        ],
        out_shape=[
            jax.ShapeDtypeStruct((n, de), jnp.float32),
            jax.ShapeDtypeStruct((n, de), jnp.float32),
        ],
    )(x, ws_t, bs.reshape(1, de), wt_t, bt.reshape(1, de))


def _make_gather(n_edges, d):
    info = plsc.get_sparse_core_info()
    nc, ns = info.num_cores, info.num_subcores
    nw = nc * ns
    b_per_w = n_edges // nw
    chunk = 80
    n_chunks = b_per_w // chunk
    n_pairs = n_chunks // 2
    dh = d // 2
    mesh = plsc.VectorSubcoreMesh(core_axis_name="c", subcore_axis_name="s")

    @functools.partial(
        pl.kernel,
        mesh=mesh,
        out_type=jax.ShapeDtypeStruct((n_edges, dh), jnp.int32),
        scratch_types=[
            pltpu.VMEM((chunk,), jnp.int32),
            pltpu.VMEM((chunk,), jnp.int32),
            pltpu.VMEM((chunk, d), jnp.float32),
            pltpu.VMEM((chunk, d), jnp.float32),
            pltpu.VMEM((chunk, dh), jnp.int32),
            pltpu.SemaphoreType.DMA,
            pltpu.SemaphoreType.DMA,
            pltpu.VMEM((chunk,), jnp.int32),
            pltpu.VMEM((chunk,), jnp.int32),
            pltpu.VMEM((chunk, d), jnp.float32),
            pltpu.VMEM((chunk, d), jnp.float32),
            pltpu.VMEM((chunk, dh), jnp.int32),
            pltpu.SemaphoreType.DMA,
            pltpu.SemaphoreType.DMA,
        ],
    )
    def gather_k(xs_hbm, xt_hbm, src_hbm, dst_hbm, g_hbm,
                 ia0, ib0, rs0, rt0, pk0, ss0, st0,
                 ia1, ib1, rs1, rt1, pk1, ss1, st1):
        wid = lax.axis_index("s") * nc + lax.axis_index("c")
        base = wid * b_per_w

        def issue(ci, idx_s, idx_t, rows_s, rows_t, sem_s, sem_t):
            off = base + ci * chunk
            pltpu.sync_copy(src_hbm.at[pl.ds(off, chunk)], idx_s)
            pltpu.sync_copy(dst_hbm.at[pl.ds(off, chunk)], idx_t)
            pltpu.async_copy(xs_hbm.at[idx_s], rows_s, sem_s)
            pltpu.async_copy(xt_hbm.at[idx_t], rows_t, sem_t)

        def process(ci, idx_s, idx_t, rows_s, rows_t, packed, sem_s, sem_t):
            pltpu.make_async_copy(xs_hbm.at[idx_s], rows_s, sem_s).wait()
            pltpu.make_async_copy(xt_hbm.at[idx_t], rows_t, sem_t).wait()

            def rowbody(r, carry):
                for c in range(d // 32):
                    a = rows_s[r, pl.ds(16 * c, 16)] + rows_t[r, pl.ds(16 * c, 16)]
                    b = rows_s[r, pl.ds(dh + 16 * c, 16)] + rows_t[r, pl.ds(dh + 16 * c, 16)]
                    ai = jax.lax.bitcast_convert_type(a, jnp.uint32)
                    bi = jax.lax.bitcast_convert_type(b, jnp.uint32)
                    ar = ai + jnp.uint32(0x7FFF) + ((ai >> 16) & jnp.uint32(1))
                    br = bi + jnp.uint32(0x7FFF) + ((bi >> 16) & jnp.uint32(1))
                    w = (ar & jnp.uint32(0xFFFF0000)) | (br >> 16)
                    packed[r, pl.ds(16 * c, 16)] = jax.lax.bitcast_convert_type(w, jnp.int32)
                return carry

            lax.fori_loop(0, chunk, rowbody, 0)
            off = base + ci * chunk
            pltpu.sync_copy(packed, g_hbm.at[pl.ds(off, chunk)])

        issue(0, ia0, ib0, rs0, rt0, ss0, st0)

        def body(j, carry):
            c0 = 2 * j
            issue(c0 + 1, ia1, ib1, rs1, rt1, ss1, st1)
            process(c0, ia0, ib0, rs0, rt0, pk0, ss0, st0)
            issue(c0 + 2, ia0, ib0, rs0, rt0, ss0, st0)
            process(c0 + 1, ia1, ib1, rs1, rt1, pk1, ss1, st1)
            return carry

        lax.fori_loop(0, n_pairs, body, 0)
        process(n_chunks - 1, ia0, ib0, rs0, rt0, pk0, ss0, st0)

    return gather_k


def _unpack_bf16_pairs(w):
    wu = jax.lax.bitcast_convert_type(w, jnp.uint32)
    hi = jax.lax.bitcast_convert_type(wu & jnp.uint32(0xFFFF0000), jnp.float32)
    lo = jax.lax.bitcast_convert_type(wu << 16, jnp.float32)
    return jnp.concatenate([hi, lo], axis=1)


def _edge_body(ea_ref, g_ref, we_ref, be_ref, w1_ref, b1_ref, out_ref):
    h = jnp.dot(ea_ref[...], we_ref[...], preferred_element_type=jnp.float32)
    h = h + be_ref[...] + _unpack_bf16_pairs(g_ref[...])
    h = jnp.maximum(h, 0.0)
    out_ref[...] = jnp.dot(h, w1_ref[...], preferred_element_type=jnp.float32) + b1_ref[...]


def _edge_mlp(ea, g, we_t, be, w1_t, b1):
    e, d = ea.shape
    blk = 2560
    grid = e // blk
    return pl.pallas_call(
        _edge_body,
        grid=(grid,),
        in_specs=[
            pl.BlockSpec((blk, d), lambda i: (i, 0)),
            pl.BlockSpec((blk, d // 2), lambda i: (i, 0)),
            pl.BlockSpec((d, d), lambda i: (0, 0)),
            pl.BlockSpec((1, d), lambda i: (0, 0)),
            pl.BlockSpec((d, d), lambda i: (0, 0)),
            pl.BlockSpec((1, d), lambda i: (0, 0)),
        ],
        out_specs=pl.BlockSpec((blk, d), lambda i: (i, 0)),
        out_shape=jax.ShapeDtypeStruct((e, d), jnp.float32),
    )(ea, g, we_t, be.reshape(1, d), w1_t, b1.reshape(1, d))


def kernel(x, edge_index, edge_attr, W_e, b_e, W_s, b_s, W_t, b_t, W_1, b_1):
    src = edge_index[0].astype(jnp.int32)
    dst = edge_index[1].astype(jnp.int32)
    xs, xt = _node_transform(x, W_s.T, b_s, W_t.T, b_t)
    gather = _make_gather(edge_attr.shape[0], edge_attr.shape[1])
    g = gather(xs, xt, src, dst)
    return _edge_mlp(edge_attr, g, W_e.T, b_e, W_1.T, b_1)


# speedup vs baseline: 4.2599x; 1.3207x over previous
import functools

import jax
import jax.numpy as jnp
from jax import lax
from jax.experimental import pallas as pl
from jax.experimental.pallas import tpu as pltpu
from jax.experimental.pallas import tpu_sc as plsc


def _node_body(x_ref, ws_ref, bs_ref, wt_ref, bt_ref, xs_ref, xt_ref):
    xv = x_ref[...]
    xs_ref[...] = jnp.dot(xv, ws_ref[...], preferred_element_type=jnp.float32) + bs_ref[...]
    xt_ref[...] = jnp.dot(xv, wt_ref[...], preferred_element_type=jnp.float32) + bt_ref[...]


def _node_transform(x, ws_t, bs, wt_t, bt):
    n, d = x.shape
    de = ws_t.shape[1]
    blk = 1000
    grid = n // blk
    return pl.pallas_call(
        _node_body,
        grid=(grid,),
        in_specs=[
            pl.BlockSpec((blk, d), lambda i: (i, 0)),
            pl.BlockSpec((d, de), lambda i: (0, 0)),
            pl.BlockSpec((1, de), lambda i: (0, 0)),
            pl.BlockSpec((d, de), lambda i: (0, 0)),
            pl.BlockSpec((1, de), lambda i: (0, 0)),
        ],
        out_specs=[
            pl.BlockSpec((blk, de), lambda i: (i, 0)),
            pl.BlockSpec((blk, de), lambda i: (i, 0)),
        ],
        out_shape=[
            jax.ShapeDtypeStruct((n, de), jnp.float32),
            jax.ShapeDtypeStruct((n, de), jnp.float32),
        ],
    )(x, ws_t, bs.reshape(1, de), wt_t, bt.reshape(1, de))


def _make_gather(n_edges, d):
    info = plsc.get_sparse_core_info()
    nc, ns = info.num_cores, info.num_subcores
    nw = nc * ns
    b_per_w = n_edges // nw
    chunk = 80
    n_chunks = b_per_w // chunk
    n_pairs = (n_chunks - 1) // 2
    dh = d // 2
    mesh = plsc.VectorSubcoreMesh(core_axis_name="c", subcore_axis_name="s")

    slot = lambda: [
        pltpu.VMEM((chunk,), jnp.int32),
        pltpu.VMEM((chunk,), jnp.int32),
        pltpu.VMEM((chunk, d), jnp.float32),
        pltpu.VMEM((chunk, d), jnp.float32),
        pltpu.VMEM((chunk, dh), jnp.int32),
        pltpu.SemaphoreType.DMA,
        pltpu.SemaphoreType.DMA,
        pltpu.SemaphoreType.DMA,
        pltpu.SemaphoreType.DMA,
    ]

    @functools.partial(
        pl.kernel,
        mesh=mesh,
        out_type=jax.ShapeDtypeStruct((n_edges, dh), jnp.int32),
        scratch_types=slot() + slot(),
    )
    def gather_k(xs_hbm, xt_hbm, src_hbm, dst_hbm, g_hbm,
                 ia0, ib0, rs0, rt0, pk0, ss0, st0, sw0, si0,
                 ia1, ib1, rs1, rt1, pk1, ss1, st1, sw1, si1):
        wid = lax.axis_index("s") * nc + lax.axis_index("c")
        base = wid * b_per_w

        def idx_load(ci, ia, ib, si):
            off = base + ci * chunk
            pltpu.async_copy(src_hbm.at[pl.ds(off, chunk)], ia, si)
            pltpu.async_copy(dst_hbm.at[pl.ds(off, chunk)], ib, si)

        def issue(ci, ia, ib, rs, rt, ss, st, si):
            pltpu.make_async_copy(src_hbm.at[pl.ds(base, chunk)], ia, si).wait()
            pltpu.make_async_copy(dst_hbm.at[pl.ds(base, chunk)], ib, si).wait()
            pltpu.async_copy(xs_hbm.at[ia], rs, ss)
            pltpu.async_copy(xt_hbm.at[ib], rt, st)

        def process(ci, ia, ib, rs, rt, pk, ss, st, sw, si):
            pltpu.make_async_copy(xs_hbm.at[ia], rs, ss).wait()
            pltpu.make_async_copy(xt_hbm.at[ib], rt, st).wait()

            @pl.when(ci + 2 < n_chunks)
            def _prefetch_idx():
                idx_load(ci + 2, ia, ib, si)

            @pl.when(ci >= 2)
            def _wait_prev_wb():
                pltpu.make_async_copy(pk, g_hbm.at[pl.ds(base, chunk)], sw).wait()

            def rowbody(r, carry):
                for c in range(d // 32):
                    a = rs[r, pl.ds(16 * c, 16)] + rt[r, pl.ds(16 * c, 16)]
                    b = rs[r, pl.ds(dh + 16 * c, 16)] + rt[r, pl.ds(dh + 16 * c, 16)]
                    ai = jax.lax.bitcast_convert_type(a, jnp.uint32)
                    bi = jax.lax.bitcast_convert_type(b, jnp.uint32)
                    ar = ai + jnp.uint32(0x7FFF) + ((ai >> 16) & jnp.uint32(1))
                    br = bi + jnp.uint32(0x7FFF) + ((bi >> 16) & jnp.uint32(1))
                    w = (ar & jnp.uint32(0xFFFF0000)) | (br >> 16)
                    pk[r, pl.ds(16 * c, 16)] = jax.lax.bitcast_convert_type(w, jnp.int32)
                return carry

            lax.fori_loop(0, chunk, rowbody, 0)
            pltpu.async_copy(pk, g_hbm.at[pl.ds(base + ci * chunk, chunk)], sw)

        idx_load(0, ia0, ib0, si0)
        idx_load(1, ia1, ib1, si1)
        issue(0, ia0, ib0, rs0, rt0, ss0, st0, si0)

        def body(j, carry):
            c0 = 2 * j
            issue(c0 + 1, ia1, ib1, rs1, rt1, ss1, st1, si1)
            process(c0, ia0, ib0, rs0, rt0, pk0, ss0, st0, sw0, si0)
            issue(c0 + 2, ia0, ib0, rs0, rt0, ss0, st0, si0)
            process(c0 + 1, ia1, ib1, rs1, rt1, pk1, ss1, st1, sw1, si1)
            return carry

        lax.fori_loop(0, n_pairs, body, 0)
        process(n_chunks - 1, ia0, ib0, rs0, rt0, pk0, ss0, st0, sw0, si0)
        pltpu.make_async_copy(pk0, g_hbm.at[pl.ds(base, chunk)], sw0).wait()
        pltpu.make_async_copy(pk1, g_hbm.at[pl.ds(base, chunk)], sw1).wait()

    return gather_k


def _unpack_bf16_pairs(w):
    wu = jax.lax.bitcast_convert_type(w, jnp.uint32)
    hi = jax.lax.bitcast_convert_type(wu & jnp.uint32(0xFFFF0000), jnp.float32)
    lo = jax.lax.bitcast_convert_type(wu << 16, jnp.float32)
    return jnp.concatenate([hi, lo], axis=1)


def _edge_body(ea_ref, g_ref, we_ref, be_ref, w1_ref, b1_ref, out_ref):
    h = jnp.dot(ea_ref[...], we_ref[...], preferred_element_type=jnp.float32)
    h = h + be_ref[...] + _unpack_bf16_pairs(g_ref[...])
    h = jnp.maximum(h, 0.0)
    out_ref[...] = jnp.dot(h, w1_ref[...], preferred_element_type=jnp.float32) + b1_ref[...]


def _edge_body_prev(ea_ref, g_ref, we_ref, be_ref, w1_ref, b1_ref, prev_ref, out_ref):
    del prev_ref
    _edge_body(ea_ref, g_ref, we_ref, be_ref, w1_ref, b1_ref, out_ref)


def _edge_mlp_chunk(ea, g, we_t, be, w1_t, b1, k, ek, prev):
    e, d = ea.shape
    blk = 2560
    grid = ek // blk
    base = k * grid
    in_specs = [
        pl.BlockSpec((blk, d), lambda i, b=base: (b + i, 0)),
        pl.BlockSpec((blk, d // 2), lambda i: (i, 0)),
        pl.BlockSpec((d, d), lambda i: (0, 0)),
        pl.BlockSpec((1, d), lambda i: (0, 0)),
        pl.BlockSpec((d, d), lambda i: (0, 0)),
        pl.BlockSpec((1, d), lambda i: (0, 0)),
    ]
    args = [ea, g, we_t, be.reshape(1, d), w1_t, b1.reshape(1, d)]
    if prev is None:
        body = _edge_body
        aliases = {}
    else:
        body = _edge_body_prev
        in_specs.append(pl.BlockSpec(memory_space=pl.ANY))
        args.append(prev)
        aliases = {6: 0}
    return pl.pallas_call(
        body,
        grid=(grid,),
        in_specs=in_specs,
        out_specs=pl.BlockSpec((blk, d), lambda i, b=base: (b + i, 0)),
        out_shape=jax.ShapeDtypeStruct((e, d), jnp.float32),
        input_output_aliases=aliases,
    )(*args)


def kernel(x, edge_index, edge_attr, W_e, b_e, W_s, b_s, W_t, b_t, W_1, b_1):
    src = edge_index[0].astype(jnp.int32)
    dst = edge_index[1].astype(jnp.int32)
    xs, xt = _node_transform(x, W_s.T, b_s, W_t.T, b_t)
    e, d = edge_attr.shape
    n_split = 5
    ek = e // n_split
    gather = _make_gather(ek, d)
    we_t, w1_t = W_e.T, W_1.T
    out = None
    for k in range(n_split):
        sl = slice(k * ek, (k + 1) * ek)
        g_k = gather(xs, xt, src[sl], dst[sl])
        out = _edge_mlp_chunk(edge_attr, g_k, we_t, b_e, w1_t, b_1, k, ek, out)
    return out
